# baseline (speedup 1.0000x reference)
import functools

import jax
import jax.numpy as jnp
from jax import lax
from jax.experimental import pallas as pl
from jax.experimental.pallas import tpu as pltpu
from jax.experimental.pallas import tpu_sc as plsc

_NC = 2
_NS = 16
_LANE = 128
_G = 64
_NBUF = 8
_AHEAD = 4


def _rsqrt_v(x):
  i = lax.bitcast_convert_type(x, jnp.int32)
  i = jnp.int32(0x5F3759DF) - (i >> 1)
  y = lax.bitcast_convert_type(i, jnp.float32)
  xh = 0.5 * x
  y = y * (1.5 - xh * y * y)
  y = y * (1.5 - xh * y * y)
  y = y * (1.5 - xh * y * y)
  return y


def _agg_pipeline(K, g_sh, acc_sh, si_v, di_v, rows_v, semg, sems):
  for r in range(_AHEAD):
    pltpu.async_copy(g_sh.at[si_v.at[r]], rows_v.at[r], semg.at[r])

  def body(jj, carry):
    for b in range(4):
      j = 4 * jj + b
      cur = lax.rem(j, _NBUF)
      nxt = lax.rem(j + _AHEAD, _NBUF)
      pltpu.make_async_copy(g_sh.at[si_v.at[j]], rows_v.at[cur],
                            semg.at[cur]).wait()
      pltpu.async_copy(rows_v.at[cur], acc_sh.at[di_v.at[j]], sems.at[cur],
                       add=True)

      @pl.when(j >= _AHEAD)
      def _():
        pltpu.make_async_copy(rows_v.at[nxt], acc_sh.at[di_v.at[j]],
                              sems.at[nxt]).wait()

      @pl.when(j + _AHEAD < K)
      def _():
        pltpu.async_copy(g_sh.at[si_v.at[j + _AHEAD]], rows_v.at[nxt],
                         semg.at[nxt])
    return carry

  lax.fori_loop(0, K // 4, body, 0)
  for r in range(K - _AHEAD, K):
    pltpu.make_async_copy(rows_v.at[r % _NBUF], acc_sh.at[di_v.at[r]],
                          sems.at[r % _NBUF]).wait()


def _sc_deg_agg(N1p, F, K, rpt):
  mesh = plsc.VectorSubcoreMesh(core_axis_name="c", subcore_axis_name="s", num_cores=_NC, num_subcores=_NS)

  @functools.partial(
      pl.kernel,
      out_type=(jax.ShapeDtypeStruct((_NC, N1p, F), jnp.float32),
                jax.ShapeDtypeStruct((N1p, 16), jnp.float32)),
      mesh=mesh,
      scratch_types=[
          pltpu.VMEM((K, _LANE), jnp.int32),
          pltpu.VMEM((K, _LANE), jnp.int32),
          pltpu.VMEM((K, _LANE), jnp.int32),
          pltpu.VMEM((_NBUF, _LANE, F), jnp.float32),
          pltpu.VMEM((rpt, F), jnp.float32),
          pltpu.VMEM((rpt, 16), jnp.float32),
          pltpu.VMEM((rpt, 16), jnp.float32),
          pltpu.VMEM((_LANE, 16), jnp.float32),
          pltpu.VMEM_SHARED((N1p, F), jnp.float32),
          pltpu.VMEM_SHARED((N1p, F), jnp.float32),
          pltpu.VMEM_SHARED((N1p, 16), jnp.float32),
          pltpu.SemaphoreType.DMA((_NBUF,)),
          pltpu.SemaphoreType.DMA((_NBUF,)),
          pltpu.SemaphoreType.DMA((_NBUF,)),
      ],
      compiler_params=pltpu.CompilerParams(use_tc_tiling_on_sc=False),
  )
  def k(h_hbm, src_hbm, dst_hbm, ones_hbm, zeros8_hbm, zerosf_hbm,
        out_hbm, dinv_hbm,
        si_v, di_v, do_v, rows_v, h_v, dg_v, dv_v, ones_v, acc_sh, g_sh,
        deg_sh, semd, semg, sems):
    c = lax.axis_index("c")
    s = lax.axis_index("s")
    w = c * _NS + s
    w2 = (1 - c) * _NS + s
    sl = pl.ds(s * rpt, rpt)
    pltpu.sync_copy(zeros8_hbm.at[sl], deg_sh.at[sl])
    pltpu.sync_copy(dst_hbm.at[pl.ds(w * K, K)], di_v)
    pltpu.sync_copy(dst_hbm.at[pl.ds(w2 * K, K)], do_v)
    pltpu.sync_copy(src_hbm.at[pl.ds(w * K, K)], si_v)
    pltpu.sync_copy(h_hbm.at[sl], h_v)
    pltpu.sync_copy(ones_hbm, ones_v)
    plsc.subcore_barrier()

    def deg_pass(dref):
      for r in range(_NBUF):
        pltpu.async_copy(ones_v, deg_sh.at[dref.at[r]], semd.at[r], add=True)

      def deg_body(j, carry):
        q = lax.rem(j, _NBUF)
        pltpu.make_async_copy(ones_v, deg_sh.at[dref.at[j]],
                              semd.at[q]).wait()
        pltpu.async_copy(ones_v, deg_sh.at[dref.at[j + _NBUF]], semd.at[q],
                         add=True)
        return carry

      lax.fori_loop(0, K - _NBUF, deg_body, 0)
      for r in range(K - _NBUF, K):
        pltpu.make_async_copy(ones_v, deg_sh.at[dref.at[r]],
                              semd.at[r % _NBUF]).wait()

    deg_pass(di_v)
    deg_pass(do_v)
    plsc.subcore_barrier()

    pltpu.sync_copy(deg_sh.at[sl], dg_v)

    def row_body(r, carry):
      y = _rsqrt_v(1.0 + dg_v[r])
      dv_v[r] = y
      h_v[r] = h_v[r] * y
      return carry

    lax.fori_loop(0, rpt, row_body, 0)

    pltpu.sync_copy(h_v, g_sh.at[sl])

    @pl.when(c == 0)
    def _():
      pltpu.sync_copy(h_v, acc_sh.at[sl])
      pltpu.sync_copy(dv_v, dinv_hbm.at[sl])

    @pl.when(c != 0)
    def _():
      pltpu.sync_copy(zerosf_hbm.at[sl], acc_sh.at[sl])

    plsc.subcore_barrier()
    _agg_pipeline(K, g_sh, acc_sh, si_v, di_v, rows_v, semg, sems)
    plsc.subcore_barrier()
    pltpu.sync_copy(acc_sh.at[sl], out_hbm.at[c, sl])

  return k


def _sc_scatter(N1p, F, K, rpt):
  mesh = plsc.VectorSubcoreMesh(core_axis_name="c", subcore_axis_name="s", num_cores=_NC, num_subcores=_NS)

  @functools.partial(
      pl.kernel,
      out_type=jax.ShapeDtypeStruct((_NC, N1p, F), jnp.float32),
      mesh=mesh,
      scratch_types=[
          pltpu.VMEM((K, _LANE), jnp.int32),
          pltpu.VMEM((K, _LANE), jnp.int32),
          pltpu.VMEM((_NBUF, _LANE, F), jnp.float32),
          pltpu.VMEM_SHARED((N1p, F), jnp.float32),
          pltpu.VMEM_SHARED((N1p, F), jnp.float32),
          pltpu.SemaphoreType.DMA((_NBUF,)),
          pltpu.SemaphoreType.DMA((_NBUF,)),
      ],
      compiler_params=pltpu.CompilerParams(use_tc_tiling_on_sc=False),
  )
  def scat_kernel(g_hbm, src_hbm, dst_hbm, zeros_hbm, out_hbm,
                  si_v, di_v, rows_v, acc_sh, g_sh, semg, sems):
    c = lax.axis_index("c")
    s = lax.axis_index("s")
    w = c * _NS + s
    sl = pl.ds(s * rpt, rpt)
    pltpu.sync_copy(g_hbm.at[sl], g_sh.at[sl])

    @pl.when(c == 0)
    def _():
      pltpu.sync_copy(g_hbm.at[sl], acc_sh.at[sl])

    @pl.when(c != 0)
    def _():
      pltpu.sync_copy(zeros_hbm.at[sl], acc_sh.at[sl])

    pltpu.sync_copy(src_hbm.at[pl.ds(w * K, K)], si_v)
    pltpu.sync_copy(dst_hbm.at[pl.ds(w * K, K)], di_v)
    plsc.subcore_barrier()
    _agg_pipeline(K, g_sh, acc_sh, si_v, di_v, rows_v, semg, sems)
    plsc.subcore_barrier()
    pltpu.sync_copy(acc_sh.at[sl], out_hbm.at[c, sl])

  return scat_kernel


def _tc1_body(x_ref, w1_ref, h_ref):
  h_ref[...] = jnp.dot(x_ref[...], w1_ref[...],
                       preferred_element_type=jnp.float32)


def _tc2_body(sp_ref, dinv_ref, b1_ref, w2_ref, g2_ref):
  dinv = dinv_ref[:, 0:1]
  ssum = sp_ref[0] + sp_ref[1]
  a = jnp.maximum(ssum * dinv + b1_ref[...], 0.0)
  g2_ref[...] = (jnp.dot(a, w2_ref[...], preferred_element_type=jnp.float32)
                 * dinv)


def _tc3_body(sp_ref, dinv_ref, b2_ref, batch_ref, wout_ref, bout_ref,
              out_ref):
  n = batch_ref.shape[1]
  dinv = dinv_ref[:, 0:1]
  ssum = sp_ref[0] + sp_ref[1]
  a = jnp.maximum(ssum * dinv + b2_ref[...], 0.0)[:n, :]
  gids = lax.broadcasted_iota(jnp.int32, (_G, n), 0)
  mask = (batch_ref[...] == gids).astype(jnp.float32)
  sums = jnp.dot(mask, a, preferred_element_type=jnp.float32)
  counts = jnp.sum(mask, axis=1, keepdims=True)
  pooled = sums / jnp.maximum(counts, 1.0)
  logits = (jnp.dot(pooled, wout_ref[...], preferred_element_type=jnp.float32)
            + bout_ref[...])
  m = jnp.max(logits, axis=1, keepdims=True)
  z = logits - m
  out_ref[...] = z - jnp.log(jnp.sum(jnp.exp(z), axis=1, keepdims=True))


def kernel(x, edge_index, batch, W1, b1, W2, b2, Wout, bout):
  N, F_IN = x.shape
  H1 = W1.shape[1]
  H2 = W2.shape[1]
  C = Wout.shape[1]
  E = edge_index.shape[1]
  tiles = _NC * _NS
  K = ((-(-E // (tiles * _LANE)) + 7) // 8) * 8
  Ep = tiles * K * _LANE
  N1p = ((N + 1 + 127) // 128) * 128
  rpt = N1p // _NS

  src = edge_index[0].astype(jnp.int32)
  dst = edge_index[1].astype(jnp.int32)
  pad = jnp.full((Ep - E,), N, jnp.int32)
  src2 = jnp.concatenate([src, pad]).reshape(tiles * K, _LANE)
  dst2 = jnp.concatenate([dst, pad]).reshape(tiles * K, _LANE)
  ones16 = jnp.ones((_LANE, 16), jnp.float32)
  zeros16 = jnp.zeros((N1p, 16), jnp.float32)
  zeros1 = jnp.zeros((N1p, H1), jnp.float32)
  zeros2 = jnp.zeros((N1p, H2), jnp.float32)
  xpad = jnp.concatenate([x, jnp.zeros((N1p - N, F_IN), x.dtype)])
  batch2 = batch.astype(jnp.int32).reshape(1, N)

  h = pl.pallas_call(
      _tc1_body,
      out_shape=jax.ShapeDtypeStruct((N1p, H1), jnp.float32),
  )(xpad, W1)

  sp1, dinv = _sc_deg_agg(N1p, H1, K, rpt)(h, src2, dst2, ones16, zeros16,
                                           zeros1)

  g2 = pl.pallas_call(
      _tc2_body,
      out_shape=jax.ShapeDtypeStruct((N1p, H2), jnp.float32),
  )(sp1, dinv, b1.reshape(1, H1), W2)

  sp2 = _sc_scatter(N1p, H2, K, rpt)(g2, src2, dst2, zeros2)

  out = pl.pallas_call(
      _tc3_body,
      out_shape=jax.ShapeDtypeStruct((_G, C), jnp.float32),
  )(sp2, dinv, b2.reshape(1, H2), batch2, Wout, bout.reshape(1, C))
  return out


# speedup vs baseline: 1.0409x; 1.0409x over previous
import functools

import jax
import jax.numpy as jnp
from jax import lax
from jax.experimental import pallas as pl
from jax.experimental.pallas import tpu as pltpu
from jax.experimental.pallas import tpu_sc as plsc

_NC = 2
_NS = 16
_LANE = 128
_G = 64
_NBUF = 8
_AHEAD = 4


def _rsqrt_v(x):
  i = lax.bitcast_convert_type(x, jnp.int32)
  i = jnp.int32(0x5F3759DF) - (i >> 1)
  y = lax.bitcast_convert_type(i, jnp.float32)
  xh = 0.5 * x
  y = y * (1.5 - xh * y * y)
  y = y * (1.5 - xh * y * y)
  y = y * (1.5 - xh * y * y)
  return y


def _agg_pipeline(K, g_sh, acc_sh, si_v, di_v, rows_v, semg, sems):
  for r in range(_AHEAD):
    pltpu.async_copy(g_sh.at[si_v.at[r]], rows_v.at[r], semg.at[r])

  def body(jj, carry):
    for b in range(4):
      j = 4 * jj + b
      cur = lax.rem(j, _NBUF)
      nxt = lax.rem(j + _AHEAD, _NBUF)
      pltpu.make_async_copy(g_sh.at[si_v.at[j]], rows_v.at[cur],
                            semg.at[cur]).wait()
      pltpu.async_copy(rows_v.at[cur], acc_sh.at[di_v.at[j]], sems.at[cur],
                       add=True)

      @pl.when(j >= _AHEAD)
      def _():
        pltpu.make_async_copy(rows_v.at[nxt], acc_sh.at[di_v.at[j]],
                              sems.at[nxt]).wait()

      @pl.when(j + _AHEAD < K)
      def _():
        pltpu.async_copy(g_sh.at[si_v.at[j + _AHEAD]], rows_v.at[nxt],
                         semg.at[nxt])
    return carry

  lax.fori_loop(0, K // 4, body, 0)
  for r in range(K - _AHEAD, K):
    pltpu.make_async_copy(rows_v.at[r % _NBUF], acc_sh.at[di_v.at[r]],
                          sems.at[r % _NBUF]).wait()


def _sc_deg_agg(N1p, F, K, rpt):
  mesh = plsc.VectorSubcoreMesh(core_axis_name="c", subcore_axis_name="s", num_cores=_NC, num_subcores=_NS)

  @functools.partial(
      pl.kernel,
      out_type=(jax.ShapeDtypeStruct((_NC, N1p, F), jnp.float32),
                jax.ShapeDtypeStruct((N1p, 16), jnp.float32)),
      mesh=mesh,
      scratch_types=[
          pltpu.VMEM((K, _LANE), jnp.int32),
          pltpu.VMEM((K, _LANE), jnp.int32),
          pltpu.VMEM((K, _LANE), jnp.int32),
          pltpu.VMEM((_NBUF, _LANE, F), jnp.float32),
          pltpu.VMEM((rpt, F), jnp.float32),
          pltpu.VMEM((rpt, 16), jnp.float32),
          pltpu.VMEM((rpt, 16), jnp.float32),
          pltpu.VMEM((_LANE, 8), jnp.float32),
          pltpu.VMEM_SHARED((N1p, F), jnp.float32),
          pltpu.VMEM_SHARED((N1p, F), jnp.float32),
          pltpu.VMEM_SHARED((N1p, 8), jnp.float32),
          pltpu.SemaphoreType.DMA((_NBUF,)),
          pltpu.SemaphoreType.DMA((_NBUF,)),
          pltpu.SemaphoreType.DMA((_NBUF,)),
      ],
      compiler_params=pltpu.CompilerParams(use_tc_tiling_on_sc=False),
  )
  def k(h_hbm, src_hbm, dst_hbm, ones_hbm, zeros8_hbm, zerosf_hbm,
        out_hbm, dinv_hbm,
        si_v, di_v, do_v, rows_v, h_v, dg_v, dv_v, ones_v, acc_sh, g_sh,
        deg_sh, semd, semg, sems):
    c = lax.axis_index("c")
    s = lax.axis_index("s")
    w = c * _NS + s
    w2 = (1 - c) * _NS + s
    sl = pl.ds(s * rpt, rpt)
    pltpu.sync_copy(zeros8_hbm.at[sl], deg_sh.at[sl])
    pltpu.sync_copy(dst_hbm.at[pl.ds(w * K, K)], di_v)
    pltpu.sync_copy(dst_hbm.at[pl.ds(w2 * K, K)], do_v)
    pltpu.sync_copy(src_hbm.at[pl.ds(w * K, K)], si_v)
    pltpu.sync_copy(h_hbm.at[sl], h_v)
    pltpu.sync_copy(ones_hbm, ones_v)
    plsc.subcore_barrier()

    def deg_pass(dref):
      for r in range(_NBUF):
        pltpu.async_copy(ones_v, deg_sh.at[dref.at[r]], semd.at[r], add=True)

      def deg_body(j, carry):
        q = lax.rem(j, _NBUF)
        pltpu.make_async_copy(ones_v, deg_sh.at[dref.at[j]],
                              semd.at[q]).wait()
        pltpu.async_copy(ones_v, deg_sh.at[dref.at[j + _NBUF]], semd.at[q],
                         add=True)
        return carry

      lax.fori_loop(0, K - _NBUF, deg_body, 0)
      for r in range(K - _NBUF, K):
        pltpu.make_async_copy(ones_v, deg_sh.at[dref.at[r]],
                              semd.at[r % _NBUF]).wait()

    deg_pass(di_v)
    deg_pass(do_v)
    plsc.subcore_barrier()

    pltpu.sync_copy(deg_sh.at[sl], dg_v.at[pl.ds(0, rpt), pl.ds(0, 8)])
    pltpu.sync_copy(deg_sh.at[sl], dg_v.at[pl.ds(0, rpt), pl.ds(8, 8)])

    def row_body(r, carry):
      y = _rsqrt_v(1.0 + dg_v[r])
      dv_v[r] = y
      h_v[r] = h_v[r] * y
      return carry

    lax.fori_loop(0, rpt, row_body, 0)

    pltpu.sync_copy(h_v, g_sh.at[sl])

    @pl.when(c == 0)
    def _():
      pltpu.sync_copy(h_v, acc_sh.at[sl])
      pltpu.sync_copy(dv_v, dinv_hbm.at[sl])

    @pl.when(c != 0)
    def _():
      pltpu.sync_copy(zerosf_hbm.at[sl], acc_sh.at[sl])

    plsc.subcore_barrier()
    _agg_pipeline(K, g_sh, acc_sh, si_v, di_v, rows_v, semg, sems)
    plsc.subcore_barrier()
    pltpu.sync_copy(acc_sh.at[sl], out_hbm.at[c, sl])

  return k


def _sc_scatter(N1p, F, K, rpt):
  mesh = plsc.VectorSubcoreMesh(core_axis_name="c", subcore_axis_name="s", num_cores=_NC, num_subcores=_NS)

  @functools.partial(
      pl.kernel,
      out_type=jax.ShapeDtypeStruct((_NC, N1p, F), jnp.float32),
      mesh=mesh,
      scratch_types=[
          pltpu.VMEM((K, _LANE), jnp.int32),
          pltpu.VMEM((K, _LANE), jnp.int32),
          pltpu.VMEM((_NBUF, _LANE, F), jnp.float32),
          pltpu.VMEM_SHARED((N1p, F), jnp.float32),
          pltpu.VMEM_SHARED((N1p, F), jnp.float32),
          pltpu.SemaphoreType.DMA((_NBUF,)),
          pltpu.SemaphoreType.DMA((_NBUF,)),
      ],
      compiler_params=pltpu.CompilerParams(use_tc_tiling_on_sc=False),
  )
  def scat_kernel(g_hbm, src_hbm, dst_hbm, zeros_hbm, out_hbm,
                  si_v, di_v, rows_v, acc_sh, g_sh, semg, sems):
    c = lax.axis_index("c")
    s = lax.axis_index("s")
    w = c * _NS + s
    sl = pl.ds(s * rpt, rpt)
    pltpu.sync_copy(g_hbm.at[sl], g_sh.at[sl])

    @pl.when(c == 0)
    def _():
      pltpu.sync_copy(g_hbm.at[sl], acc_sh.at[sl])

    @pl.when(c != 0)
    def _():
      pltpu.sync_copy(zeros_hbm.at[sl], acc_sh.at[sl])

    pltpu.sync_copy(src_hbm.at[pl.ds(w * K, K)], si_v)
    pltpu.sync_copy(dst_hbm.at[pl.ds(w * K, K)], di_v)
    plsc.subcore_barrier()
    _agg_pipeline(K, g_sh, acc_sh, si_v, di_v, rows_v, semg, sems)
    plsc.subcore_barrier()
    pltpu.sync_copy(acc_sh.at[sl], out_hbm.at[c, sl])

  return scat_kernel


def _tc1_body(x_ref, w1_ref, h_ref):
  n = x_ref.shape[0]
  h_ref[pl.ds(0, n), :] = jnp.dot(x_ref[...], w1_ref[...],
                                  preferred_element_type=jnp.float32)
  h_ref[pl.ds(n, h_ref.shape[0] - n), :] = jnp.zeros(
      (h_ref.shape[0] - n, w1_ref.shape[1]), jnp.float32)


def _tc2_body(sp_ref, dinv_ref, b1_ref, w2_ref, g2_ref):
  dinv = dinv_ref[:, 0:1]
  ssum = sp_ref[0] + sp_ref[1]
  a = jnp.maximum(ssum * dinv + b1_ref[...], 0.0)
  g2_ref[...] = (jnp.dot(a, w2_ref[...], preferred_element_type=jnp.float32)
                 * dinv)


def _tc3_body(sp_ref, dinv_ref, b2_ref, batch_ref, wout_ref, bout_ref,
              out_ref):
  n = batch_ref.shape[1]
  dinv = dinv_ref[:, 0:1]
  ssum = sp_ref[0] + sp_ref[1]
  a = jnp.maximum(ssum * dinv + b2_ref[...], 0.0)[:n, :]
  gids = lax.broadcasted_iota(jnp.int32, (_G, n), 0)
  mask = (batch_ref[...] == gids).astype(jnp.float32)
  sums = jnp.dot(mask, a, preferred_element_type=jnp.float32)
  counts = jnp.sum(mask, axis=1, keepdims=True)
  pooled = sums / jnp.maximum(counts, 1.0)
  logits = (jnp.dot(pooled, wout_ref[...], preferred_element_type=jnp.float32)
            + bout_ref[...])
  m = jnp.max(logits, axis=1, keepdims=True)
  z = logits - m
  out_ref[...] = z - jnp.log(jnp.sum(jnp.exp(z), axis=1, keepdims=True))


def kernel(x, edge_index, batch, W1, b1, W2, b2, Wout, bout):
  N, F_IN = x.shape
  H1 = W1.shape[1]
  H2 = W2.shape[1]
  C = Wout.shape[1]
  E = edge_index.shape[1]
  tiles = _NC * _NS
  K = ((-(-E // (tiles * _LANE)) + 7) // 8) * 8
  Ep = tiles * K * _LANE
  N1p = ((N + 1 + 127) // 128) * 128
  rpt = N1p // _NS

  src = edge_index[0].astype(jnp.int32)
  dst = edge_index[1].astype(jnp.int32)
  pad = jnp.full((Ep - E,), N, jnp.int32)
  src2 = jnp.concatenate([src, pad]).reshape(tiles * K, _LANE)
  dst2 = jnp.concatenate([dst, pad]).reshape(tiles * K, _LANE)
  ones8 = jnp.ones((_LANE, 8), jnp.float32)
  zeros8 = jnp.zeros((N1p, 8), jnp.float32)
  zeros1 = jnp.zeros((N1p, H1), jnp.float32)
  zeros2 = jnp.zeros((N1p, H2), jnp.float32)
  batch2 = batch.astype(jnp.int32).reshape(1, N)

  h = pl.pallas_call(
      _tc1_body,
      out_shape=jax.ShapeDtypeStruct((N1p, H1), jnp.float32),
  )(x, W1)

  sp1, dinv = _sc_deg_agg(N1p, H1, K, rpt)(h, src2, dst2, ones8, zeros8,
                                           zeros1)

  g2 = pl.pallas_call(
      _tc2_body,
      out_shape=jax.ShapeDtypeStruct((N1p, H2), jnp.float32),
  )(sp1, dinv, b1.reshape(1, H1), W2)

  sp2 = _sc_scatter(N1p, H2, K, rpt)(g2, src2, dst2, zeros2)

  out = pl.pallas_call(
      _tc3_body,
      out_shape=jax.ShapeDtypeStruct((_G, C), jnp.float32),
  )(sp2, dinv, b2.reshape(1, H2), batch2, Wout, bout.reshape(1, C))
  return out
